# baseline (speedup 1.0000x reference)
import functools

import jax
import jax.numpy as jnp
from jax import lax
from jax.experimental import pallas as pl
from jax.experimental.pallas import tpu as pltpu
from jax.experimental.pallas import tpu_sc as plsc

_BS = 64
_L = 16


def _sc_body(catT_hbm, out_hbm, src_v, tile_v, sem):
    n2d, w = catT_hbm.shape
    cpw = src_v.shape[0]
    hw = tile_v.shape[1]
    groups = hw // _L
    nchunk = n2d // cpw
    bpw = _BS // (32 // nchunk)
    nc = lax.axis_index("c")
    ns = lax.axis_index("s")
    wid = ns * 2 + nc
    q = wid % nchunk
    b0 = (wid // nchunk) * bpw
    woff = pl.multiple_of(q * cpw, cpw)
    pltpu.sync_copy(catT_hbm.at[pl.ds(woff, cpw)], src_v)

    iota = lax.broadcasted_iota(jnp.int32, (_L,), 0)
    qv = jnp.zeros((_L,), jnp.int32) + q
    is_top = qv < (nchunk // 2)

    def build(i, carry):
        cc = i // groups
        g = i % groups
        lane = g * _L + iota
        col_idx = jnp.where(is_top, lane % w, lane // w)
        row_idx = jnp.zeros((_L,), jnp.int32) + cc
        val = plsc.load_gather(src_v, [row_idx, col_idx])
        plsc.store_scatter(tile_v, [row_idx, lane], val)
        return carry

    lax.fori_loop(0, cpw * groups, build, 0)

    copies = [
        pltpu.make_async_copy(tile_v, out_hbm.at[b0 + i, pl.ds(woff, cpw)], sem)
        for i in range(bpw)
    ]
    for cp in copies:
        cp.start()
    for cp in copies:
        cp.wait()


def kernel(row_embed, col_embed, bs):
    h, d = row_embed.shape
    w = col_embed.shape[0]
    catT = jnp.concatenate([col_embed.T, row_embed.T], axis=0)
    cpw = 64
    sck = pl.kernel(
        _sc_body,
        out_type=jax.ShapeDtypeStruct((_BS, 2 * d, h * w), jnp.float32),
        mesh=plsc.VectorSubcoreMesh(core_axis_name="c", subcore_axis_name="s"),
        scratch_types=[
            pltpu.VMEM((cpw, w), jnp.float32),
            pltpu.VMEM((cpw, h * w), jnp.float32),
            pltpu.SemaphoreType.DMA,
        ],
        compiler_params=pltpu.CompilerParams(
            use_tc_tiling_on_sc=True, needs_layout_passes=False),
    )
    out = sck(catT)
    return out.reshape(_BS, 2 * d, h, w)


# speedup vs baseline: 1.1668x; 1.1668x over previous
import jax
import jax.numpy as jnp
from jax.experimental import pallas as pl
from jax.experimental.pallas import tpu as pltpu

_BS = 64


def _body(colT_ref, rowT_ref, o_ref, pos):
    d, w = colT_ref.shape
    h = rowT_ref.shape[1]

    @pl.when(pl.program_id(0) == 0)
    def _():
        colT = colT_ref[...]
        for y in range(h):
            pos[0:d, y * w:(y + 1) * w] = colT
            pos[d:2 * d, y * w:(y + 1) * w] = jnp.broadcast_to(
                rowT_ref[:, y:y + 1], (d, w))

    o_ref[0] = pos[...]


def kernel(row_embed, col_embed, bs):
    h, d = row_embed.shape
    w = col_embed.shape[0]
    colT = col_embed.T
    rowT = row_embed.T
    out = pl.pallas_call(
        _body,
        grid=(_BS,),
        in_specs=[
            pl.BlockSpec((d, w), lambda b: (0, 0)),
            pl.BlockSpec((d, h), lambda b: (0, 0)),
        ],
        out_specs=pl.BlockSpec((1, 2 * d, h * w), lambda b: (b, 0, 0)),
        out_shape=jax.ShapeDtypeStruct((_BS, 2 * d, h * w), jnp.float32),
        scratch_shapes=[pltpu.VMEM((2 * d, h * w), jnp.float32)],
    )(colT, rowT)
    return out.reshape(_BS, 2 * d, h, w)


# speedup vs baseline: 1.2092x; 1.0363x over previous
import jax
import jax.numpy as jnp
from jax.experimental import pallas as pl
from jax.experimental.pallas import tpu as pltpu

_BS = 64


def _body(colT_ref, rowT_ref, o_hbm, pos, sem0, sem1):
    d, w = colT_ref.shape
    h = rowT_ref.shape[1]
    colT = colT_ref[...]
    for y in range(h):
        pos[0:d, y * w:(y + 1) * w] = colT
        pos[d:2 * d, y * w:(y + 1) * w] = jnp.broadcast_to(
            rowT_ref[:, y:y + 1], (d, w))
    sems = [sem0, sem1]
    copies = [
        pltpu.make_async_copy(pos, o_hbm.at[b], sems[b % 2]) for b in range(_BS)
    ]
    for b, c in enumerate(copies):
        c.start(priority=b % 2)
    for c in copies:
        c.wait()


def kernel(row_embed, col_embed, bs):
    h, d = row_embed.shape
    w = col_embed.shape[0]
    colT = col_embed.T
    rowT = row_embed.T
    out = pl.pallas_call(
        _body,
        in_specs=[
            pl.BlockSpec((d, w), lambda: (0, 0)),
            pl.BlockSpec((d, h), lambda: (0, 0)),
        ],
        out_specs=pl.BlockSpec(memory_space=pl.ANY),
        out_shape=jax.ShapeDtypeStruct((_BS, 2 * d, h * w), jnp.float32),
        scratch_shapes=[
            pltpu.VMEM((2 * d, h * w), jnp.float32),
            pltpu.SemaphoreType.DMA,
            pltpu.SemaphoreType.DMA,
        ],
    )(colT, rowT)
    return out.reshape(_BS, 2 * d, h, w)
